# baseline (speedup 1.0000x reference)
import jax
import jax.numpy as jnp
from jax.experimental import pallas as pl
from jax.experimental.pallas import tpu as pltpu

_SPLIT = 12


def _pos_kernel(row_ref, col_ref, out1_ref, out2_ref, scratch, sems):
    b1 = out1_ref.shape[0]
    b2 = out2_ref.shape[0]
    h, d = row_ref.shape
    w = col_ref.shape[0]
    ce = col_ref[:, :]
    re = row_ref[:, :]

    def _dma(i, q):
        dst = out1_ref.at[i] if i < b1 else out2_ref.at[i - b1]
        if q == 0:
            return pltpu.make_async_copy(
                scratch.at[:, :, :d], dst.at[:, :, :d], sems.at[i])
        return pltpu.make_async_copy(
            scratch.at[:, :, d:], dst.at[:, :, d:], sems.at[b1 + b2 + i])

    scratch[:, :, :d] = jnp.broadcast_to(ce[None, :, :], (h, w, d))
    for i in range(b1 + b2):
        _dma(i, 0).start()
    scratch[:, :, d:] = jnp.broadcast_to(re[:, None, :], (h, w, d))
    for i in range(b1 + b2):
        _dma(i, 1).start()
    for i in range(b1 + b2):
        _dma(i, 0).wait()
        _dma(i, 1).wait()


def kernel(x, row_embed, col_embed):
    b = x.shape[0]
    h, w = x.shape[-2], x.shape[-1]
    d = row_embed.shape[-1]
    b1 = min(_SPLIT, b)
    b2 = b - b1
    out1, out2 = pl.pallas_call(
        _pos_kernel,
        in_specs=[
            pl.BlockSpec((h, d), lambda: (0, 0)),
            pl.BlockSpec((w, d), lambda: (0, 0)),
        ],
        out_specs=[
            pl.BlockSpec(memory_space=pl.ANY),
            pl.BlockSpec(memory_space=pl.ANY),
        ],
        out_shape=[
            jax.ShapeDtypeStruct((b1, h, w, 2 * d), row_embed.dtype),
            jax.ShapeDtypeStruct((b2, h, w, 2 * d), row_embed.dtype),
        ],
        scratch_shapes=[
            pltpu.VMEM((h, w, 2 * d), row_embed.dtype),
            pltpu.SemaphoreType.DMA((2 * b,)),
        ],
    )(row_embed[:h], col_embed[:w])
    out = jnp.concatenate([out1, out2], axis=0)
    return jnp.transpose(out, (0, 3, 1, 2))


# speedup vs baseline: 2.5798x; 2.5798x over previous
import jax
import jax.numpy as jnp
from jax.experimental import pallas as pl
from jax.experimental.pallas import tpu as pltpu


def _pos_kernel(row_ref, col_ref, out_ref, scratch, sems):
    b = out_ref.shape[0]
    h, d = row_ref.shape
    w = col_ref.shape[0]
    ce = col_ref[:, :]
    re = row_ref[:, :]
    scratch[:, :, :d] = jnp.broadcast_to(ce[None, :, :], (h, w, d))
    for i in range(b):
        pltpu.make_async_copy(
            scratch.at[:, :, :d], out_ref.at[i, :, :, :d], sems.at[i]).start()
    scratch[:, :, d:] = jnp.broadcast_to(re[:, None, :], (h, w, d))
    for i in range(b):
        pltpu.make_async_copy(
            scratch.at[:, :, d:], out_ref.at[i, :, :, d:], sems.at[b + i]).start()
    for i in range(b):
        pltpu.make_async_copy(
            scratch.at[:, :, :d], out_ref.at[i, :, :, :d], sems.at[i]).wait()
        pltpu.make_async_copy(
            scratch.at[:, :, d:], out_ref.at[i, :, :, d:], sems.at[b + i]).wait()


def kernel(x, row_embed, col_embed):
    b = x.shape[0]
    h, w = x.shape[-2], x.shape[-1]
    d = row_embed.shape[-1]
    out = pl.pallas_call(
        _pos_kernel,
        in_specs=[
            pl.BlockSpec((h, d), lambda: (0, 0)),
            pl.BlockSpec((w, d), lambda: (0, 0)),
        ],
        out_specs=pl.BlockSpec(memory_space=pl.ANY),
        out_shape=jax.ShapeDtypeStruct((b, h, w, 2 * d), row_embed.dtype),
        scratch_shapes=[
            pltpu.VMEM((h, w, 2 * d), row_embed.dtype),
            pltpu.SemaphoreType.DMA((2 * b,)),
        ],
    )(row_embed[:h], col_embed[:w])
    return jnp.transpose(out, (0, 3, 1, 2))


# speedup vs baseline: 2.9894x; 1.1587x over previous
import jax
import jax.numpy as jnp
from jax.experimental import pallas as pl
from jax.experimental.pallas import tpu as pltpu


def _pos_kernel(row_ref, col_ref, out_ref, scratch, sems):
    b, h, w, two_d = out_ref.shape
    d = two_d // 2
    ce = col_ref[:w, :]
    re = row_ref[:h, :]
    scratch[:, :, :d] = jnp.broadcast_to(ce[None, :, :], (h, w, d))
    for i in range(b):
        pltpu.make_async_copy(
            scratch.at[:, :, :d], out_ref.at[i, :, :, :d], sems.at[i]).start()
    scratch[:, :, d:] = jnp.broadcast_to(re[:, None, :], (h, w, d))
    for i in range(b):
        pltpu.make_async_copy(
            scratch.at[:, :, d:], out_ref.at[i, :, :, d:], sems.at[b + i]).start()
    for i in range(b):
        pltpu.make_async_copy(
            scratch.at[:, :, :d], out_ref.at[i, :, :, :d], sems.at[i]).wait()
        pltpu.make_async_copy(
            scratch.at[:, :, d:], out_ref.at[i, :, :, d:], sems.at[b + i]).wait()


def kernel(x, row_embed, col_embed):
    b = x.shape[0]
    h, w = x.shape[-2], x.shape[-1]
    d = row_embed.shape[-1]
    out = pl.pallas_call(
        _pos_kernel,
        in_specs=[
            pl.BlockSpec(row_embed.shape, lambda: (0, 0)),
            pl.BlockSpec(col_embed.shape, lambda: (0, 0)),
        ],
        out_specs=pl.BlockSpec(memory_space=pl.ANY),
        out_shape=jax.ShapeDtypeStruct((b, h, w, 2 * d), row_embed.dtype),
        scratch_shapes=[
            pltpu.VMEM((h, w, 2 * d), row_embed.dtype),
            pltpu.SemaphoreType.DMA((2 * b,)),
        ],
    )(row_embed, col_embed)
    return jnp.transpose(out, (0, 3, 1, 2))


# speedup vs baseline: 2.9999x; 1.0035x over previous
import jax
import jax.numpy as jnp
from jax.experimental import pallas as pl
from jax.experimental.pallas import tpu as pltpu


def _pos_kernel(row_ref, col_ref, out_ref, scratch, sems):
    b, h, w, two_d = out_ref.shape
    d = two_d // 2
    ce = col_ref[:w, :]
    re = row_ref[:h, :]
    scratch[:, :, :d] = jnp.broadcast_to(ce[None, :, :], (h, w, d))
    for i in range(b):
        pltpu.make_async_copy(
            scratch.at[:, :, :d], out_ref.at[i, :, :, :d], sems.at[i]).start(priority=i % 2)
    scratch[:, :, d:] = jnp.broadcast_to(re[:, None, :], (h, w, d))
    for i in range(b):
        pltpu.make_async_copy(
            scratch.at[:, :, d:], out_ref.at[i, :, :, d:], sems.at[b + i]).start(priority=i % 2)
    for i in range(b):
        pltpu.make_async_copy(
            scratch.at[:, :, :d], out_ref.at[i, :, :, :d], sems.at[i]).wait()
        pltpu.make_async_copy(
            scratch.at[:, :, d:], out_ref.at[i, :, :, d:], sems.at[b + i]).wait()


def kernel(x, row_embed, col_embed):
    b = x.shape[0]
    h, w = x.shape[-2], x.shape[-1]
    d = row_embed.shape[-1]
    out = pl.pallas_call(
        _pos_kernel,
        in_specs=[
            pl.BlockSpec(row_embed.shape, lambda: (0, 0)),
            pl.BlockSpec(col_embed.shape, lambda: (0, 0)),
        ],
        out_specs=pl.BlockSpec(memory_space=pl.ANY),
        out_shape=jax.ShapeDtypeStruct((b, h, w, 2 * d), row_embed.dtype),
        scratch_shapes=[
            pltpu.VMEM((h, w, 2 * d), row_embed.dtype),
            pltpu.SemaphoreType.DMA((2 * b,)),
        ],
    )(row_embed, col_embed)
    return jnp.transpose(out, (0, 3, 1, 2))
